# baseline (speedup 1.0000x reference)
import functools

import jax
import jax.numpy as jnp
from jax.experimental import pallas as pl
from jax.experimental.pallas import tpu as pltpu

_R = 4096
_S = 4
_C = _R // _S
_E = 256
_EPS = 1e-10
_COMMIT = 0.25


def _in_copies(x_hbm, xbuf, sems, t, slot):
    return [
        pltpu.make_async_copy(
            x_hbm.at[pl.ds(t * _R + i * _C, _C), :],
            xbuf.at[slot, pl.ds(i * _C, _C), :],
            sems.at[slot, i],
        )
        for i in range(_S)
    ]


def _out_copies(buf, hbm, sems, t, slot):
    return [
        pltpu.make_async_copy(
            buf.at[slot, pl.ds(i * _C, _C), :],
            hbm.at[pl.ds(t * _R + i * _C, _C), :],
            sems.at[slot, i],
        )
        for i in range(_S)
    ]


def _vq_main(x_hbm, w_ref, wt_ref,
             qst_hbm, enc_hbm, idx_hbm, hist_hbm, sse_hbm,
             xbuf, qstbuf, encbuf, idxbuf, hist_acc, sse_acc,
             in_sems, qst_sems, enc_sems, idx_sems, fin_sems):
    t = pl.program_id(0)
    num = pl.num_programs(0)
    slot = jax.lax.rem(t, 2)

    @pl.when(t == 0)
    def _prologue():
        for c in _in_copies(x_hbm, xbuf, in_sems, 0, 0):
            c.start()
        hist_acc[...] = jnp.zeros_like(hist_acc)
        sse_acc[...] = jnp.zeros_like(sse_acc)

    @pl.when(t + 1 < num)
    def _prefetch():
        for c in _in_copies(x_hbm, xbuf, in_sems, t + 1, 1 - slot):
            c.start()

    for c in _in_copies(x_hbm, xbuf, in_sems, t, slot):
        c.wait()

    @pl.when(t >= 2)
    def _drain_prev():
        for c in _out_copies(qstbuf, qst_hbm, qst_sems, t - 2, slot):
            c.wait()
        for c in _out_copies(encbuf, enc_hbm, enc_sems, t - 2, slot):
            c.wait()
        pltpu.make_async_copy(
            idxbuf.at[slot], idx_hbm.at[pl.ds((t - 2) * _R, _R), :],
            idx_sems.at[slot]).wait()

    x = xbuf[slot]
    w = w_ref[...]
    m2w_b = (-2.0 * w).astype(jnp.bfloat16)
    xw_m2 = jnp.dot(x.astype(jnp.bfloat16), m2w_b,
                    preferred_element_type=jnp.float32)
    w2 = jnp.sum(w * w, axis=0, keepdims=True)
    d = w2 + xw_m2

    dmin = jnp.min(d, axis=1, keepdims=True)
    lane = jax.lax.broadcasted_iota(jnp.int32, (_R, _E), 1).astype(jnp.float32)
    m = d == dmin
    enc = jnp.where(m, 1.0, 0.0)
    idx = jnp.min(jnp.where(m, lane, _E), axis=1, keepdims=True)

    enc_b = enc.astype(jnp.bfloat16)
    q = jnp.dot(enc_b, wt_ref[...].astype(jnp.bfloat16),
                preferred_element_type=jnp.float32)
    qstbuf[slot] = q
    encbuf[slot] = enc
    idxbuf[slot] = idx.astype(jnp.int32)

    ones_b = jnp.ones((1, _R), jnp.bfloat16)
    hist_acc[...] += jnp.dot(ones_b, enc_b, preferred_element_type=jnp.float32)
    sse_acc[...] += jnp.broadcast_to(jnp.sum(x * x) + jnp.sum(dmin), (1, 128))

    for c in _out_copies(qstbuf, qst_hbm, qst_sems, t, slot):
        c.start()
    for c in _out_copies(encbuf, enc_hbm, enc_sems, t, slot):
        c.start()
    pltpu.make_async_copy(
        idxbuf.at[slot], idx_hbm.at[pl.ds(t * _R, _R), :],
        idx_sems.at[slot]).start()

    @pl.when(t == num - 1)
    def _epilogue():
        @pl.when(num >= 2)
        def _():
            for c in _out_copies(qstbuf, qst_hbm, qst_sems, t - 1, 1 - slot):
                c.wait()
            for c in _out_copies(encbuf, enc_hbm, enc_sems, t - 1, 1 - slot):
                c.wait()
            pltpu.make_async_copy(
                idxbuf.at[1 - slot], idx_hbm.at[pl.ds((t - 1) * _R, _R), :],
                idx_sems.at[1 - slot]).wait()
        for c in _out_copies(qstbuf, qst_hbm, qst_sems, t, slot):
            c.wait()
        for c in _out_copies(encbuf, enc_hbm, enc_sems, t, slot):
            c.wait()
        pltpu.make_async_copy(
            idxbuf.at[slot], idx_hbm.at[pl.ds(t * _R, _R), :],
            idx_sems.at[slot]).wait()
        hcopy = pltpu.make_async_copy(hist_acc, hist_hbm, fin_sems.at[0])
        scopy = pltpu.make_async_copy(sse_acc, sse_hbm, fin_sems.at[1])
        hcopy.start()
        scopy.start()
        hcopy.wait()
        scopy.wait()


def _vq_finalize(hist_ref, sse_ref, loss_ref, perp_ref, *, n_rows):
    sse = sse_ref[0, 0]
    mse = sse / (n_rows * _E)
    loss_ref[...] = jnp.broadcast_to(mse + _COMMIT * mse, (1, 1))
    p = hist_ref[...] / n_rows
    ent = -jnp.sum(p * jnp.log(p + _EPS), keepdims=True)
    perp_ref[...] = jnp.exp(ent).reshape(1, 1)


def kernel(x, w, is_training):
    lead_shape = x.shape[:-1]
    xf = x.reshape(-1, _E)
    n = xf.shape[0]
    grid = n // _R

    qst, enc, idx, hist_t, sse_t = pl.pallas_call(
        _vq_main,
        grid=(grid,),
        in_specs=[
            pl.BlockSpec(memory_space=pl.ANY),
            pl.BlockSpec((_E, _E), lambda t: (0, 0)),
            pl.BlockSpec((_E, _E), lambda t: (0, 0)),
        ],
        out_specs=[
            pl.BlockSpec(memory_space=pl.ANY),
            pl.BlockSpec(memory_space=pl.ANY),
            pl.BlockSpec(memory_space=pl.ANY),
            pl.BlockSpec(memory_space=pl.ANY),
            pl.BlockSpec(memory_space=pl.ANY),
        ],
        out_shape=[
            jax.ShapeDtypeStruct((n, _E), jnp.float32),
            jax.ShapeDtypeStruct((n, _E), jnp.float32),
            jax.ShapeDtypeStruct((n, 1), jnp.int32),
            jax.ShapeDtypeStruct((1, _E), jnp.float32),
            jax.ShapeDtypeStruct((1, 128), jnp.float32),
        ],
        scratch_shapes=[
            pltpu.VMEM((2, _R, _E), jnp.float32),
            pltpu.VMEM((2, _R, _E), jnp.float32),
            pltpu.VMEM((2, _R, _E), jnp.float32),
            pltpu.VMEM((2, _R, 1), jnp.int32),
            pltpu.VMEM((1, _E), jnp.float32),
            pltpu.VMEM((1, 128), jnp.float32),
            pltpu.SemaphoreType.DMA((2, _S)),
            pltpu.SemaphoreType.DMA((2, _S)),
            pltpu.SemaphoreType.DMA((2, _S)),
            pltpu.SemaphoreType.DMA((2,)),
            pltpu.SemaphoreType.DMA((2,)),
        ],
    )(xf, w, w.T)

    loss, perp = pl.pallas_call(
        functools.partial(_vq_finalize, n_rows=n),
        out_shape=[
            jax.ShapeDtypeStruct((1, 1), jnp.float32),
            jax.ShapeDtypeStruct((1, 1), jnp.float32),
        ],
    )(hist_t, sse_t)

    return (qst.reshape(x.shape), loss[0, 0], perp[0, 0], enc,
            idx.reshape(lead_shape))


# speedup vs baseline: 1.0446x; 1.0446x over previous
import functools

import jax
import jax.numpy as jnp
from jax.experimental import pallas as pl
from jax.experimental.pallas import tpu as pltpu

_R = 4096
_NIN = 4
_S = 4
_C = _R // _S
_E = 256
_EPS = 1e-10
_COMMIT = 0.25


def _in_copies(x_hbm, xbuf, sems, t, slot):
    return [
        pltpu.make_async_copy(
            x_hbm.at[pl.ds(t * _R + i * _C, _C), :],
            xbuf.at[slot, pl.ds(i * _C, _C), :],
            sems.at[slot, i],
        )
        for i in range(_S)
    ]


def _out_copies(buf, hbm, sems, t, slot):
    return [
        pltpu.make_async_copy(
            buf.at[slot, pl.ds(i * _C, _C), :],
            hbm.at[pl.ds(t * _R + i * _C, _C), :],
            sems.at[slot, i],
        )
        for i in range(_S)
    ]


def _vq_main(x_hbm, w_ref, wt_ref,
             qst_hbm, enc_hbm, idx_hbm, hist_hbm, sse_hbm,
             xbuf, qstbuf, encbuf, idxbuf, hist_acc, sse_acc,
             in_sems, qst_sems, enc_sems, idx_sems, fin_sems):
    t = pl.program_id(0)
    num = pl.num_programs(0)
    slot = jax.lax.rem(t, 2)
    islot = jax.lax.rem(t, _NIN)

    @pl.when(t == 0)
    def _prologue():
        for u in range(_NIN - 1):
            for c in _in_copies(x_hbm, xbuf, in_sems, u, u):
                c.start()
        hist_acc[...] = jnp.zeros_like(hist_acc)
        sse_acc[...] = jnp.zeros_like(sse_acc)

    @pl.when(t + _NIN - 1 < num)
    def _prefetch():
        for c in _in_copies(x_hbm, xbuf, in_sems, t + _NIN - 1,
                            jax.lax.rem(t + _NIN - 1, _NIN)):
            c.start()

    for c in _in_copies(x_hbm, xbuf, in_sems, t, islot):
        c.wait()

    @pl.when(t >= 2)
    def _drain_prev():
        for c in _out_copies(qstbuf, qst_hbm, qst_sems, t - 2, slot):
            c.wait()
        for c in _out_copies(encbuf, enc_hbm, enc_sems, t - 2, slot):
            c.wait()
        pltpu.make_async_copy(
            idxbuf.at[slot], idx_hbm.at[pl.ds((t - 2) * _R, _R), :],
            idx_sems.at[slot]).wait()

    x = xbuf[islot]
    w = w_ref[...]
    m2w_b = (-2.0 * w).astype(jnp.bfloat16)
    xw_m2 = jnp.dot(x.astype(jnp.bfloat16), m2w_b,
                    preferred_element_type=jnp.float32)
    w2 = jnp.sum(w * w, axis=0, keepdims=True)
    d = w2 + xw_m2

    dmin = jnp.min(d, axis=1, keepdims=True)
    lane = jax.lax.broadcasted_iota(jnp.int32, (_R, _E), 1).astype(jnp.float32)
    m = d == dmin
    enc = jnp.where(m, 1.0, 0.0)
    idx = jnp.min(jnp.where(m, lane, _E), axis=1, keepdims=True)

    enc_b = enc.astype(jnp.bfloat16)
    q = jnp.dot(enc_b, wt_ref[...].astype(jnp.bfloat16),
                preferred_element_type=jnp.float32)
    qstbuf[slot] = q
    encbuf[slot] = enc
    idxbuf[slot] = idx.astype(jnp.int32)

    ones_b = jnp.ones((1, _R), jnp.bfloat16)
    hist_acc[...] += jnp.dot(ones_b, enc_b, preferred_element_type=jnp.float32)
    sse_acc[...] += jnp.broadcast_to(jnp.sum(x * x) + jnp.sum(dmin), (1, 128))

    for c in _out_copies(qstbuf, qst_hbm, qst_sems, t, slot):
        c.start()
    for c in _out_copies(encbuf, enc_hbm, enc_sems, t, slot):
        c.start()
    pltpu.make_async_copy(
        idxbuf.at[slot], idx_hbm.at[pl.ds(t * _R, _R), :],
        idx_sems.at[slot]).start()

    @pl.when(t == num - 1)
    def _epilogue():
        @pl.when(num >= 2)
        def _():
            for c in _out_copies(qstbuf, qst_hbm, qst_sems, t - 1, 1 - slot):
                c.wait()
            for c in _out_copies(encbuf, enc_hbm, enc_sems, t - 1, 1 - slot):
                c.wait()
            pltpu.make_async_copy(
                idxbuf.at[1 - slot], idx_hbm.at[pl.ds((t - 1) * _R, _R), :],
                idx_sems.at[1 - slot]).wait()
        for c in _out_copies(qstbuf, qst_hbm, qst_sems, t, slot):
            c.wait()
        for c in _out_copies(encbuf, enc_hbm, enc_sems, t, slot):
            c.wait()
        pltpu.make_async_copy(
            idxbuf.at[slot], idx_hbm.at[pl.ds(t * _R, _R), :],
            idx_sems.at[slot]).wait()
        hcopy = pltpu.make_async_copy(hist_acc, hist_hbm, fin_sems.at[0])
        scopy = pltpu.make_async_copy(sse_acc, sse_hbm, fin_sems.at[1])
        hcopy.start()
        scopy.start()
        hcopy.wait()
        scopy.wait()


def _vq_finalize(hist_ref, sse_ref, loss_ref, perp_ref, *, n_rows):
    sse = sse_ref[0, 0]
    mse = sse / (n_rows * _E)
    loss_ref[...] = jnp.broadcast_to(mse + _COMMIT * mse, (1, 1))
    p = hist_ref[...] / n_rows
    ent = -jnp.sum(p * jnp.log(p + _EPS), keepdims=True)
    perp_ref[...] = jnp.exp(ent).reshape(1, 1)


def kernel(x, w, is_training):
    lead_shape = x.shape[:-1]
    xf = x.reshape(-1, _E)
    n = xf.shape[0]
    grid = n // _R

    qst, enc, idx, hist_t, sse_t = pl.pallas_call(
        _vq_main,
        grid=(grid,),
        in_specs=[
            pl.BlockSpec(memory_space=pl.ANY),
            pl.BlockSpec((_E, _E), lambda t: (0, 0)),
            pl.BlockSpec((_E, _E), lambda t: (0, 0)),
        ],
        out_specs=[
            pl.BlockSpec(memory_space=pl.ANY),
            pl.BlockSpec(memory_space=pl.ANY),
            pl.BlockSpec(memory_space=pl.ANY),
            pl.BlockSpec(memory_space=pl.ANY),
            pl.BlockSpec(memory_space=pl.ANY),
        ],
        out_shape=[
            jax.ShapeDtypeStruct((n, _E), jnp.float32),
            jax.ShapeDtypeStruct((n, _E), jnp.float32),
            jax.ShapeDtypeStruct((n, 1), jnp.int32),
            jax.ShapeDtypeStruct((1, _E), jnp.float32),
            jax.ShapeDtypeStruct((1, 128), jnp.float32),
        ],
        scratch_shapes=[
            pltpu.VMEM((_NIN, _R, _E), jnp.float32),
            pltpu.VMEM((2, _R, _E), jnp.float32),
            pltpu.VMEM((2, _R, _E), jnp.float32),
            pltpu.VMEM((2, _R, 1), jnp.int32),
            pltpu.VMEM((1, _E), jnp.float32),
            pltpu.VMEM((1, 128), jnp.float32),
            pltpu.SemaphoreType.DMA((_NIN, _S)),
            pltpu.SemaphoreType.DMA((2, _S)),
            pltpu.SemaphoreType.DMA((2, _S)),
            pltpu.SemaphoreType.DMA((2,)),
            pltpu.SemaphoreType.DMA((2,)),
        ],
    )(xf, w, w.T)

    loss, perp = pl.pallas_call(
        functools.partial(_vq_finalize, n_rows=n),
        out_shape=[
            jax.ShapeDtypeStruct((1, 1), jnp.float32),
            jax.ShapeDtypeStruct((1, 1), jnp.float32),
        ],
    )(hist_t, sse_t)

    return (qst.reshape(x.shape), loss[0, 0], perp[0, 0], enc,
            idx.reshape(lead_shape))


# speedup vs baseline: 1.0643x; 1.0188x over previous
import functools

import jax
import jax.numpy as jnp
from jax.experimental import pallas as pl
from jax.experimental.pallas import tpu as pltpu

_R = 4096
_NIN = 4
_S = 4
_K = 2
_C = _R // _S
_E = 256
_EPS = 1e-10
_COMMIT = 0.25


def _in_copies(x_hbm, xbuf, sems, t, slot):
    return [
        pltpu.make_async_copy(
            x_hbm.at[pl.ds(t * _R + i * _C, _C), :],
            xbuf.at[slot, pl.ds(i * _C, _C), :],
            sems.at[slot, i],
        )
        for i in range(_S)
    ]


def _out_copies(buf, hbm, sems, t, slot):
    return [
        pltpu.make_async_copy(
            buf.at[slot, pl.ds(i * _C, _C), :],
            hbm.at[pl.ds(t * _R + i * _C, _C), :],
            sems.at[slot, i],
        )
        for i in range(_S)
    ]


def _vq_main(x_hbm, w_ref, wt_ref,
             qst_hbm, enc_hbm, idx_hbm, hist_hbm, sse_hbm,
             xbuf, qstbuf, encbuf, idxbuf, hist_acc, sse_acc,
             in_sems, qst_sems, enc_sems, idx_sems, fin_sems):
    t = pl.program_id(0)
    num = pl.num_programs(0)
    slot = jax.lax.rem(t, 2)
    islot = jax.lax.rem(t, _NIN)

    @pl.when(t == 0)
    def _prologue():
        for u in range(_NIN - 1):
            for c in _in_copies(x_hbm, xbuf, in_sems, u, u):
                c.start()
        hist_acc[...] = jnp.zeros_like(hist_acc)
        sse_acc[...] = jnp.zeros_like(sse_acc)

    @pl.when(t + _NIN - 1 < num)
    def _prefetch():
        for c in _in_copies(x_hbm, xbuf, in_sems, t + _NIN - 1,
                            jax.lax.rem(t + _NIN - 1, _NIN)):
            c.start()

    @pl.when(t >= 2)
    def _drain_prev():
        for c in _out_copies(qstbuf, qst_hbm, qst_sems, t - 2, slot):
            c.wait()
        for c in _out_copies(encbuf, enc_hbm, enc_sems, t - 2, slot):
            c.wait()
        pltpu.make_async_copy(
            idxbuf.at[slot], idx_hbm.at[pl.ds((t - 2) * _R, _R), :],
            idx_sems.at[slot]).wait()

    w = w_ref[...]
    m2w_b = (-2.0 * w).astype(jnp.bfloat16)
    w2 = jnp.sum(w * w, axis=0, keepdims=True)
    wt_b = wt_ref[...].astype(jnp.bfloat16)
    in_cs = _in_copies(x_hbm, xbuf, in_sems, t, islot)
    per = _S // _K
    for k in range(_K):
        for c in in_cs[k * per:(k + 1) * per]:
            c.wait()
        rows = pl.ds(k * (_R // _K), _R // _K)
        x = xbuf[islot, rows, :]
        xw_m2 = jnp.dot(x.astype(jnp.bfloat16), m2w_b,
                        preferred_element_type=jnp.float32)
        d = w2 + xw_m2
        dmin = jnp.min(d, axis=1, keepdims=True)
        lane = jax.lax.broadcasted_iota(
            jnp.int32, (_R // _K, _E), 1).astype(jnp.float32)
        m = d == dmin
        enc = jnp.where(m, 1.0, 0.0)
        idx = jnp.min(jnp.where(m, lane, _E), axis=1, keepdims=True)
        enc_b = enc.astype(jnp.bfloat16)
        q = jnp.dot(enc_b, wt_b,
                    preferred_element_type=jnp.float32)
        qstbuf[slot, rows, :] = q
        encbuf[slot, rows, :] = enc
        idxbuf[slot, rows, :] = idx.astype(jnp.int32)
        ones_b = jnp.ones((1, _R // _K), jnp.bfloat16)
        hist_acc[...] += jnp.dot(ones_b, enc_b,
                                 preferred_element_type=jnp.float32)
        sse_acc[...] += jnp.broadcast_to(
            jnp.sum(x * x) + jnp.sum(dmin), (1, 128))
        for c in _out_copies(qstbuf, qst_hbm, qst_sems, t, slot)[k * per:(k + 1) * per]:
            c.start()
        for c in _out_copies(encbuf, enc_hbm, enc_sems, t, slot)[k * per:(k + 1) * per]:
            c.start()
    pltpu.make_async_copy(
        idxbuf.at[slot], idx_hbm.at[pl.ds(t * _R, _R), :],
        idx_sems.at[slot]).start()

    @pl.when(t == num - 1)
    def _epilogue():
        @pl.when(num >= 2)
        def _():
            for c in _out_copies(qstbuf, qst_hbm, qst_sems, t - 1, 1 - slot):
                c.wait()
            for c in _out_copies(encbuf, enc_hbm, enc_sems, t - 1, 1 - slot):
                c.wait()
            pltpu.make_async_copy(
                idxbuf.at[1 - slot], idx_hbm.at[pl.ds((t - 1) * _R, _R), :],
                idx_sems.at[1 - slot]).wait()
        for c in _out_copies(qstbuf, qst_hbm, qst_sems, t, slot):
            c.wait()
        for c in _out_copies(encbuf, enc_hbm, enc_sems, t, slot):
            c.wait()
        pltpu.make_async_copy(
            idxbuf.at[slot], idx_hbm.at[pl.ds(t * _R, _R), :],
            idx_sems.at[slot]).wait()
        hcopy = pltpu.make_async_copy(hist_acc, hist_hbm, fin_sems.at[0])
        scopy = pltpu.make_async_copy(sse_acc, sse_hbm, fin_sems.at[1])
        hcopy.start()
        scopy.start()
        hcopy.wait()
        scopy.wait()


def _vq_finalize(hist_ref, sse_ref, loss_ref, perp_ref, *, n_rows):
    sse = sse_ref[0, 0]
    mse = sse / (n_rows * _E)
    loss_ref[...] = jnp.broadcast_to(mse + _COMMIT * mse, (1, 1))
    p = hist_ref[...] / n_rows
    ent = -jnp.sum(p * jnp.log(p + _EPS), keepdims=True)
    perp_ref[...] = jnp.exp(ent).reshape(1, 1)


def kernel(x, w, is_training):
    lead_shape = x.shape[:-1]
    xf = x.reshape(-1, _E)
    n = xf.shape[0]
    grid = n // _R

    qst, enc, idx, hist_t, sse_t = pl.pallas_call(
        _vq_main,
        grid=(grid,),
        in_specs=[
            pl.BlockSpec(memory_space=pl.ANY),
            pl.BlockSpec((_E, _E), lambda t: (0, 0)),
            pl.BlockSpec((_E, _E), lambda t: (0, 0)),
        ],
        out_specs=[
            pl.BlockSpec(memory_space=pl.ANY),
            pl.BlockSpec(memory_space=pl.ANY),
            pl.BlockSpec(memory_space=pl.ANY),
            pl.BlockSpec(memory_space=pl.ANY),
            pl.BlockSpec(memory_space=pl.ANY),
        ],
        out_shape=[
            jax.ShapeDtypeStruct((n, _E), jnp.float32),
            jax.ShapeDtypeStruct((n, _E), jnp.float32),
            jax.ShapeDtypeStruct((n, 1), jnp.int32),
            jax.ShapeDtypeStruct((1, _E), jnp.float32),
            jax.ShapeDtypeStruct((1, 128), jnp.float32),
        ],
        scratch_shapes=[
            pltpu.VMEM((_NIN, _R, _E), jnp.float32),
            pltpu.VMEM((2, _R, _E), jnp.float32),
            pltpu.VMEM((2, _R, _E), jnp.float32),
            pltpu.VMEM((2, _R, 1), jnp.int32),
            pltpu.VMEM((1, _E), jnp.float32),
            pltpu.VMEM((1, 128), jnp.float32),
            pltpu.SemaphoreType.DMA((_NIN, _S)),
            pltpu.SemaphoreType.DMA((2, _S)),
            pltpu.SemaphoreType.DMA((2, _S)),
            pltpu.SemaphoreType.DMA((2,)),
            pltpu.SemaphoreType.DMA((2,)),
        ],
    )(xf, w, w.T)

    loss, perp = pl.pallas_call(
        functools.partial(_vq_finalize, n_rows=n),
        out_shape=[
            jax.ShapeDtypeStruct((1, 1), jnp.float32),
            jax.ShapeDtypeStruct((1, 1), jnp.float32),
        ],
    )(hist_t, sse_t)

    return (qst.reshape(x.shape), loss[0, 0], perp[0, 0], enc,
            idx.reshape(lead_shape))


# speedup vs baseline: 1.0645x; 1.0002x over previous
import functools

import jax
import jax.numpy as jnp
from jax.experimental import pallas as pl
from jax.experimental.pallas import tpu as pltpu

_R = 4096
_NIN = 4
_S = 4
_K = 2
_C = _R // _S
_E = 256
_EPS = 1e-10
_COMMIT = 0.25


def _in_copies(x_hbm, xbuf, sems, t, slot):
    return [
        pltpu.make_async_copy(
            x_hbm.at[pl.ds(t * _R + i * _C, _C), :],
            xbuf.at[slot, pl.ds(i * _C, _C), :],
            sems.at[slot, i],
        )
        for i in range(_S)
    ]


def _out_copies(buf, hbm, sems, t, slot):
    return [
        pltpu.make_async_copy(
            buf.at[slot, pl.ds(i * _C, _C), :],
            hbm.at[pl.ds(t * _R + i * _C, _C), :],
            sems.at[slot, i],
        )
        for i in range(_S)
    ]


def _vq_main(x_hbm, w_ref, wt_ref,
             qst_hbm, enc_hbm, idx_hbm, hist_hbm, sse_hbm,
             xbuf, qstbuf, encbuf, idxbuf, hist_acc, sse_acc,
             in_sems, qst_sems, enc_sems, idx_sems, fin_sems):
    t = pl.program_id(0)
    num = pl.num_programs(0)
    slot = jax.lax.rem(t, 2)
    islot = jax.lax.rem(t, _NIN)

    @pl.when(t == 0)
    def _prologue():
        for u in range(_NIN - 1):
            for c in _in_copies(x_hbm, xbuf, in_sems, u, u):
                c.start()
        hist_acc[...] = jnp.zeros_like(hist_acc)
        sse_acc[...] = jnp.zeros_like(sse_acc)

    @pl.when(t + _NIN - 1 < num)
    def _prefetch():
        for c in _in_copies(x_hbm, xbuf, in_sems, t + _NIN - 1,
                            jax.lax.rem(t + _NIN - 1, _NIN)):
            c.start()

    @pl.when(t >= 2)
    def _drain_prev():
        for c in _out_copies(qstbuf, qst_hbm, qst_sems, t - 2, slot):
            c.wait()
        for c in _out_copies(encbuf, enc_hbm, enc_sems, t - 2, slot):
            c.wait()
        pltpu.make_async_copy(
            idxbuf.at[slot], idx_hbm.at[pl.ds((t - 2) * _R, _R), :],
            idx_sems.at[slot]).wait()

    w = w_ref[...]
    w_b = w.astype(jnp.bfloat16)
    w2 = jnp.sum(w * w, axis=0, keepdims=True)
    wt_b = wt_ref[...].astype(jnp.bfloat16)
    in_cs = _in_copies(x_hbm, xbuf, in_sems, t, islot)
    per = _S // _K
    for k in range(_K):
        for c in in_cs[k * per:(k + 1) * per]:
            c.wait()
        rows = pl.ds(k * (_R // _K), _R // _K)
        x = xbuf[islot, rows, :]
        xw = jnp.dot(x.astype(jnp.bfloat16), w_b,
                     preferred_element_type=jnp.float32)
        x2 = jnp.sum(x * x, axis=1, keepdims=True)
        d = (x2 - 2.0 * xw) + w2
        dmin = jnp.min(d, axis=1, keepdims=True)
        lane = jax.lax.broadcasted_iota(
            jnp.int32, (_R // _K, _E), 1).astype(jnp.float32)
        m = d == dmin
        idx = jnp.min(jnp.where(m, lane, _E), axis=1, keepdims=True)
        enc = jnp.where(lane == idx, 1.0, 0.0)
        enc_b = enc.astype(jnp.bfloat16)
        q = jnp.dot(enc_b, wt_b,
                    preferred_element_type=jnp.float32)
        qstbuf[slot, rows, :] = q
        encbuf[slot, rows, :] = enc
        idxbuf[slot, rows, :] = idx.astype(jnp.int32)
        ones_b = jnp.ones((1, _R // _K), jnp.bfloat16)
        hist_acc[...] += jnp.dot(ones_b, enc_b,
                                 preferred_element_type=jnp.float32)
        sse_acc[...] += jnp.broadcast_to(jnp.sum(dmin), (1, 128))
        for c in _out_copies(qstbuf, qst_hbm, qst_sems, t, slot)[k * per:(k + 1) * per]:
            c.start()
        for c in _out_copies(encbuf, enc_hbm, enc_sems, t, slot)[k * per:(k + 1) * per]:
            c.start()
    pltpu.make_async_copy(
        idxbuf.at[slot], idx_hbm.at[pl.ds(t * _R, _R), :],
        idx_sems.at[slot]).start()

    @pl.when(t == num - 1)
    def _epilogue():
        @pl.when(num >= 2)
        def _():
            for c in _out_copies(qstbuf, qst_hbm, qst_sems, t - 1, 1 - slot):
                c.wait()
            for c in _out_copies(encbuf, enc_hbm, enc_sems, t - 1, 1 - slot):
                c.wait()
            pltpu.make_async_copy(
                idxbuf.at[1 - slot], idx_hbm.at[pl.ds((t - 1) * _R, _R), :],
                idx_sems.at[1 - slot]).wait()
        for c in _out_copies(qstbuf, qst_hbm, qst_sems, t, slot):
            c.wait()
        for c in _out_copies(encbuf, enc_hbm, enc_sems, t, slot):
            c.wait()
        pltpu.make_async_copy(
            idxbuf.at[slot], idx_hbm.at[pl.ds(t * _R, _R), :],
            idx_sems.at[slot]).wait()
        hcopy = pltpu.make_async_copy(hist_acc, hist_hbm, fin_sems.at[0])
        scopy = pltpu.make_async_copy(sse_acc, sse_hbm, fin_sems.at[1])
        hcopy.start()
        scopy.start()
        hcopy.wait()
        scopy.wait()


def _vq_finalize(hist_ref, sse_ref, loss_ref, perp_ref, *, n_rows):
    sse = sse_ref[0, 0]
    mse = sse / (n_rows * _E)
    loss_ref[...] = jnp.broadcast_to(mse + _COMMIT * mse, (1, 1))
    p = hist_ref[...] / n_rows
    ent = -jnp.sum(p * jnp.log(p + _EPS), keepdims=True)
    perp_ref[...] = jnp.exp(ent).reshape(1, 1)


def kernel(x, w, is_training):
    lead_shape = x.shape[:-1]
    xf = x.reshape(-1, _E)
    n = xf.shape[0]
    grid = n // _R

    qst, enc, idx, hist_t, sse_t = pl.pallas_call(
        _vq_main,
        grid=(grid,),
        in_specs=[
            pl.BlockSpec(memory_space=pl.ANY),
            pl.BlockSpec((_E, _E), lambda t: (0, 0)),
            pl.BlockSpec((_E, _E), lambda t: (0, 0)),
        ],
        out_specs=[
            pl.BlockSpec(memory_space=pl.ANY),
            pl.BlockSpec(memory_space=pl.ANY),
            pl.BlockSpec(memory_space=pl.ANY),
            pl.BlockSpec(memory_space=pl.ANY),
            pl.BlockSpec(memory_space=pl.ANY),
        ],
        out_shape=[
            jax.ShapeDtypeStruct((n, _E), jnp.float32),
            jax.ShapeDtypeStruct((n, _E), jnp.float32),
            jax.ShapeDtypeStruct((n, 1), jnp.int32),
            jax.ShapeDtypeStruct((1, _E), jnp.float32),
            jax.ShapeDtypeStruct((1, 128), jnp.float32),
        ],
        scratch_shapes=[
            pltpu.VMEM((_NIN, _R, _E), jnp.float32),
            pltpu.VMEM((2, _R, _E), jnp.float32),
            pltpu.VMEM((2, _R, _E), jnp.float32),
            pltpu.VMEM((2, _R, 1), jnp.int32),
            pltpu.VMEM((1, _E), jnp.float32),
            pltpu.VMEM((1, 128), jnp.float32),
            pltpu.SemaphoreType.DMA((_NIN, _S)),
            pltpu.SemaphoreType.DMA((2, _S)),
            pltpu.SemaphoreType.DMA((2, _S)),
            pltpu.SemaphoreType.DMA((2,)),
            pltpu.SemaphoreType.DMA((2,)),
        ],
    )(xf, w, w.T)

    loss, perp = pl.pallas_call(
        functools.partial(_vq_finalize, n_rows=n),
        out_shape=[
            jax.ShapeDtypeStruct((1, 1), jnp.float32),
            jax.ShapeDtypeStruct((1, 1), jnp.float32),
        ],
    )(hist_t, sse_t)

    return (qst.reshape(x.shape), loss[0, 0], perp[0, 0], enc,
            idx.reshape(lead_shape))
